# Initial kernel scaffold; baseline (speedup 1.0000x reference)
import jax
import jax.numpy as jnp
from jax.experimental import pallas as pl


def kernel(x, W_gate, b_gate, W_proj, b_proj):
    raise NotImplementedError("write your pallas kernel here")


# speedup vs baseline: 1.5169x; 1.5169x over previous
import functools

import jax
import jax.numpy as jnp
from jax.experimental import pallas as pl

DIM = 4096
NUM_EXPERTS = 64
HEAD = 4
H = NUM_EXPERTS * HEAD
TOP_K = 8
TOKENS = 4 * 2048
BT = 512


def _tc_body(x_ref, wg_ref, bg_ref, wp_ref, bp_ref, scores_ref, routed_ref):
    h = jnp.dot(x_ref[...], wg_ref[...], preferred_element_type=jnp.float32)
    h = h + bg_ref[...]
    h = 0.5 * h * (1.0 + jax.lax.erf(h * 0.7071067811865476))
    logits = jnp.dot(h, wp_ref[...], preferred_element_type=jnp.float32)
    logits = logits + bp_ref[...]
    m = jnp.max(logits, axis=-1, keepdims=True)
    e = jnp.exp(logits - m)
    scores = e / jnp.sum(e, axis=-1, keepdims=True)
    scores_ref[...] = scores

    idx = jax.lax.broadcasted_iota(jnp.int32, scores.shape, 1)
    work = scores
    acc = jnp.zeros_like(scores)
    for _ in range(TOP_K):
        cur = jnp.max(work, axis=-1, keepdims=True)
        cand = jnp.where(work == cur, idx, NUM_EXPERTS)
        first = idx == jnp.min(cand, axis=-1, keepdims=True)
        acc = jnp.where(first, 1.0, acc)
        work = jnp.where(first, -jnp.inf, work)
    routed_ref[...] = acc


@jax.jit
def _gate(x2d, W_gate, b_gate, W_proj, b_proj):
    grid = TOKENS // BT
    scores, routed = pl.pallas_call(
        _tc_body,
        grid=(grid,),
        in_specs=[
            pl.BlockSpec((BT, DIM), lambda i: (i, 0)),
            pl.BlockSpec((DIM, H), lambda i: (0, 0)),
            pl.BlockSpec((1, H), lambda i: (0, 0)),
            pl.BlockSpec((H, NUM_EXPERTS), lambda i: (0, 0)),
            pl.BlockSpec((1, NUM_EXPERTS), lambda i: (0, 0)),
        ],
        out_specs=[
            pl.BlockSpec((BT, NUM_EXPERTS), lambda i: (i, 0)),
            pl.BlockSpec((BT, NUM_EXPERTS), lambda i: (i, 0)),
        ],
        out_shape=[
            jax.ShapeDtypeStruct((TOKENS, NUM_EXPERTS), jnp.float32),
            jax.ShapeDtypeStruct((TOKENS, NUM_EXPERTS), jnp.float32),
        ],
    )(x2d, W_gate, b_gate.reshape(1, H), W_proj, b_proj.reshape(1, NUM_EXPERTS))
    return scores, routed


def kernel(x, W_gate, b_gate, W_proj, b_proj):
    B, T, _ = x.shape
    scores, routed = _gate(x.reshape(B * T, DIM), W_gate, b_gate, W_proj, b_proj)
    gate_scores = scores.reshape(B, T, NUM_EXPERTS)
    routed_experts = routed.reshape(B, T, NUM_EXPERTS)
    return (gate_scores, routed_experts, jnp.float32(0.0))


# speedup vs baseline: 1.7188x; 1.1331x over previous
import functools

import jax
import jax.numpy as jnp
from jax.experimental import pallas as pl

DIM = 4096
NUM_EXPERTS = 64
HEAD = 4
H = NUM_EXPERTS * HEAD
TOP_K = 8
TOKENS = 4 * 2048
BT = 1024


def _tc_body(x_ref, wg_ref, bg_ref, wp_ref, bp_ref, scores_ref, routed_ref):
    h = jnp.dot(x_ref[...], wg_ref[...], preferred_element_type=jnp.float32)
    h = h + bg_ref[...]
    h = 0.5 * h * (1.0 + jax.lax.erf(h * 0.7071067811865476))
    logits = jnp.dot(h, wp_ref[...], preferred_element_type=jnp.float32)
    logits = logits + bp_ref[...]
    m = jnp.max(logits, axis=-1, keepdims=True)
    e = jnp.exp(logits - m)
    scores = e / jnp.sum(e, axis=-1, keepdims=True)
    scores_ref[...] = scores

    idx = jax.lax.broadcasted_iota(jnp.int32, scores.shape, 1)
    work = scores
    acc = jnp.zeros_like(scores)
    for _ in range(TOP_K):
        cur = jnp.max(work, axis=-1, keepdims=True)
        cand = jnp.where(work == cur, idx, NUM_EXPERTS)
        first = idx == jnp.min(cand, axis=-1, keepdims=True)
        acc = jnp.where(first, 1.0, acc)
        work = jnp.where(first, -jnp.inf, work)
    routed_ref[...] = acc


@jax.jit
def _gate(x2d, W_gate, b_gate, W_proj, b_proj):
    grid = TOKENS // BT
    scores, routed = pl.pallas_call(
        _tc_body,
        grid=(grid,),
        in_specs=[
            pl.BlockSpec((BT, DIM), lambda i: (i, 0)),
            pl.BlockSpec((DIM, H), lambda i: (0, 0)),
            pl.BlockSpec((1, H), lambda i: (0, 0)),
            pl.BlockSpec((H, NUM_EXPERTS), lambda i: (0, 0)),
            pl.BlockSpec((1, NUM_EXPERTS), lambda i: (0, 0)),
        ],
        out_specs=[
            pl.BlockSpec((BT, NUM_EXPERTS), lambda i: (i, 0)),
            pl.BlockSpec((BT, NUM_EXPERTS), lambda i: (i, 0)),
        ],
        out_shape=[
            jax.ShapeDtypeStruct((TOKENS, NUM_EXPERTS), jnp.float32),
            jax.ShapeDtypeStruct((TOKENS, NUM_EXPERTS), jnp.float32),
        ],
    )(x2d, W_gate, b_gate.reshape(1, H), W_proj, b_proj.reshape(1, NUM_EXPERTS))
    return scores, routed


def kernel(x, W_gate, b_gate, W_proj, b_proj):
    B, T, _ = x.shape
    scores, routed = _gate(x.reshape(B * T, DIM), W_gate, b_gate, W_proj, b_proj)
    gate_scores = scores.reshape(B, T, NUM_EXPERTS)
    routed_experts = routed.reshape(B, T, NUM_EXPERTS)
    return (gate_scores, routed_experts, jnp.float32(0.0))
